# Initial kernel scaffold; baseline (speedup 1.0000x reference)
import jax
import jax.numpy as jnp
from jax.experimental import pallas as pl


def kernel(pos, buffer, mic_pos):
    raise NotImplementedError("write your pallas kernel here")


# speedup vs baseline: 5.1567x; 5.1567x over previous
import functools

import jax
import jax.numpy as jnp
from jax import lax
from jax.experimental import pallas as pl
from jax.experimental.pallas import tpu as pltpu
from jax.experimental.pallas import tpu_sc as plsc

_FS = 48000.0
_C = 343000.0
_N_MICS = 192
_OVERLAP = 2048
_WINDOW = 8192
_DK = _FS / (_C * 16.0)

_NC = 2
_NS = 16
_NW = _NC * _NS
_MPW = _N_MICS // _NW
_MARGIN = 16
_WSTART = _OVERLAP - _MARGIN
_WLEN = _WINDOW + _MARGIN
_NCHUNK = _WINDOW // 16
_NGRP = _N_MICS // 16


def _vsqrt(x):
    xi = lax.bitcast_convert_type(x, jnp.int32)
    y = lax.bitcast_convert_type((xi >> 1) + 0x1FBD1DF5, jnp.float32)
    for _ in range(3):
        y = 0.5 * (y + x / y)
    return y


def _sc_beamform(pos16, buffer, mic16):
    mesh = plsc.VectorSubcoreMesh(core_axis_name="c", subcore_axis_name="s")

    @functools.partial(
        pl.kernel,
        out_type=jax.ShapeDtypeStruct((_NW, _WINDOW), jnp.float32),
        mesh=mesh,
        compiler_params=pltpu.CompilerParams(
            use_tc_tiling_on_sc=False, needs_layout_passes=False),
        scratch_types=[
            pltpu.VMEM((3, 16), jnp.float32),
            pltpu.VMEM((3, _NGRP, 16), jnp.float32),
            pltpu.VMEM((_N_MICS,), jnp.float32),
            pltpu.VMEM((_MPW * _WLEN,), jnp.float32),
            pltpu.VMEM((_WINDOW,), jnp.float32),
        ],
    )
    def sc_kernel(pos_hbm, buf_hbm, mic_hbm, out_hbm,
                  pos_v, mic_v, dist_v, rows_v, acc_v):
        wid = lax.axis_index("s") * _NC + lax.axis_index("c")

        pltpu.sync_copy(pos_hbm, pos_v)
        pltpu.sync_copy(mic_hbm, mic_v)

        px, py, pz = pos_v[0], pos_v[1], pos_v[2]
        dmax = None
        for g in range(_NGRP):
            dx = mic_v[0, g] - px
            dy = mic_v[1, g] - py
            dz = mic_v[2, g] - pz
            d = _vsqrt(dx * dx + dy * dy + dz * dz)
            dist_v[pl.ds(g * 16, 16)] = d
            dmax = d if dmax is None else jnp.maximum(dmax, d)
        dmax_vec = jnp.full((16,), jnp.max(dmax), jnp.float32)

        lane = lax.iota(jnp.int32, 16)
        zeros16 = jnp.zeros((16,), jnp.int32)
        base1 = []
        s0 = []
        s1 = []
        for k in range(_MPW):
            m = wid * _MPW + k
            pltpu.sync_copy(
                buf_hbm.at[m, pl.ds(_WSTART, _WLEN)],
                rows_v.at[pl.ds(k * _WLEN, _WLEN)],
            )
            dvec = plsc.load_gather(dist_v, [zeros16 + m])
            delay = (dmax_vec - dvec) * _DK
            di = delay.astype(jnp.int32)
            di = jnp.minimum(jnp.maximum(di, 0), _MARGIN - 1)
            df = delay - di.astype(jnp.float32)
            base1.append(lane + (k * _WLEN + _MARGIN) - di)
            s1.append((1.0 - df) * (1.0 / _N_MICS))
            s0.append(df * (1.0 / _N_MICS))

        def chunk(i, _):
            off = i * 16
            acc = jnp.zeros((16,), jnp.float32)
            for k in range(_MPW):
                idx1 = base1[k] + off
                x1 = plsc.load_gather(rows_v, [idx1])
                x0 = plsc.load_gather(rows_v, [idx1 - 1])
                acc = acc + x1 * s1[k] + x0 * s0[k]
            acc_v[pl.ds(off, 16)] = acc
            return _

        lax.fori_loop(0, _NCHUNK, chunk, None)
        pltpu.sync_copy(acc_v, out_hbm.at[wid])

    return sc_kernel(pos16, buffer, mic16)


def _combine(parts):
    def body(x_ref, o_ref):
        o_ref[...] = jnp.sum(x_ref[...], axis=0)

    return pl.pallas_call(
        body,
        out_shape=jax.ShapeDtypeStruct((_WINDOW,), jnp.float32),
    )(parts)


def kernel(pos, buffer, mic_pos):
    pos16 = jnp.broadcast_to(pos.reshape(3, 1), (3, 16))
    mic16 = mic_pos.T.reshape(3, _NGRP, 16)
    parts = _sc_beamform(pos16, buffer, mic16)
    return _combine(parts)


# speedup vs baseline: 7.2315x; 1.4023x over previous
import functools

import jax
import jax.numpy as jnp
from jax import lax
from jax.experimental import pallas as pl
from jax.experimental.pallas import tpu as pltpu
from jax.experimental.pallas import tpu_sc as plsc

_FS = 48000.0
_C = 343000.0
_N_MICS = 192
_OVERLAP = 2048
_WINDOW = 8192
_DK = _FS / (_C * 16.0)

_NC = 2
_NS = 16
_MPW = 8
_NW = _N_MICS // _MPW
_WSTART = 1920
_MARGIN = _OVERLAP - _WSTART
_WLEN = _WINDOW + _MARGIN
_SPLIT = 4224
_NCHUNK = _WINDOW // 16
_HALF = _NCHUNK // 2
_NGRP = _N_MICS // 16


def _vsqrt(x):
    xi = lax.bitcast_convert_type(x, jnp.int32)
    y = lax.bitcast_convert_type((xi >> 1) + 0x1FBD1DF5, jnp.float32)
    for _ in range(3):
        y = 0.5 * (y + x / y)
    return y


def _sc_beamform(geom, buffer):
    mesh = plsc.VectorSubcoreMesh(core_axis_name="c", subcore_axis_name="s")

    @functools.partial(
        pl.kernel,
        out_type=jax.ShapeDtypeStruct((_NW, _WINDOW), jnp.float32),
        mesh=mesh,
        compiler_params=pltpu.CompilerParams(needs_layout_passes=False),
        scratch_types=[
            pltpu.VMEM((8, 256), jnp.float32),
            pltpu.VMEM((_N_MICS,), jnp.float32),
            pltpu.VMEM((_MPW, _WLEN), jnp.float32),
            pltpu.VMEM((_WINDOW,), jnp.float32),
            pltpu.SemaphoreType.DMA,
            pltpu.SemaphoreType.DMA,
        ],
    )
    def sc_kernel(geom_hbm, buf_hbm, out_hbm,
                  geom_v, dist_v, rows_v, acc_v, sem1, sem2):
        wid = lax.axis_index("s") * _NC + lax.axis_index("c")

        @pl.when(wid < _NW)
        def _():
            row0 = pl.multiple_of(wid * _MPW, _MPW)
            cp1 = pltpu.async_copy(
                buf_hbm.at[pl.ds(row0, _MPW), pl.ds(_WSTART, _SPLIT)],
                rows_v.at[:, pl.ds(0, _SPLIT)], sem1)
            cp2 = pltpu.async_copy(
                buf_hbm.at[pl.ds(row0, _MPW),
                           pl.ds(_WSTART + _SPLIT, _WLEN - _SPLIT)],
                rows_v.at[:, pl.ds(_SPLIT, _WLEN - _SPLIT)], sem2)

            pltpu.sync_copy(geom_hbm, geom_v)

            px = geom_v[3, pl.ds(0, 16)]
            py = geom_v[4, pl.ds(0, 16)]
            pz = geom_v[5, pl.ds(0, 16)]
            dmax = None
            for g in range(_NGRP):
                dx = geom_v[0, pl.ds(g * 16, 16)] - px
                dy = geom_v[1, pl.ds(g * 16, 16)] - py
                dz = geom_v[2, pl.ds(g * 16, 16)] - pz
                d = _vsqrt(dx * dx + dy * dy + dz * dz)
                dist_v[pl.ds(g * 16, 16)] = d
                dmax = d if dmax is None else jnp.maximum(dmax, d)
            dmax_vec = jnp.full((16,), jnp.max(dmax), jnp.float32)

            lane = lax.iota(jnp.int32, 16)
            zeros16 = jnp.zeros((16,), jnp.int32)
            rowk, base1, base0, s0, s1 = [], [], [], [], []
            for k in range(_MPW):
                m = wid * _MPW + k
                dvec = plsc.load_gather(dist_v, [zeros16 + m])
                delay = (dmax_vec - dvec) * _DK
                di = delay.astype(jnp.int32)
                di = jnp.minimum(jnp.maximum(di, 0), _MARGIN - 1)
                df = delay - di.astype(jnp.float32)
                b1 = lane + _MARGIN - di
                rowk.append(zeros16 + k)
                base1.append(b1)
                base0.append(b1 - 1)
                s1.append((1.0 - df) * (1.0 / _N_MICS))
                s0.append(df * (1.0 / _N_MICS))

            def chunk(i, _):
                off = i * 16
                acc = jnp.zeros((16,), jnp.float32)
                for k in range(_MPW):
                    x1 = plsc.load_gather(rows_v, [rowk[k], base1[k] + off])
                    x0 = plsc.load_gather(rows_v, [rowk[k], base0[k] + off])
                    acc = acc + x1 * s1[k] + x0 * s0[k]
                acc_v[pl.ds(off, 16)] = acc

            cp1.wait()
            lax.fori_loop(0, _HALF, chunk, None)
            cp2.wait()
            lax.fori_loop(_HALF, _NCHUNK, chunk, None)

            pltpu.sync_copy(acc_v, out_hbm.at[wid])

    return sc_kernel(geom, buffer)


def _combine(parts):
    def body(x_ref, o_ref):
        o_ref[...] = jnp.sum(x_ref[...], axis=0)

    return pl.pallas_call(
        body,
        out_shape=jax.ShapeDtypeStruct((_WINDOW,), jnp.float32),
    )(parts)


def kernel(pos, buffer, mic_pos):
    geom = jnp.zeros((8, 256), jnp.float32)
    geom = geom.at[0:3, 0:_N_MICS].set(mic_pos.T)
    geom = geom.at[3:6, :].set(jnp.broadcast_to(pos.reshape(3, 1), (3, 256)))
    parts = _sc_beamform(geom, buffer)
    return _combine(parts)
